# baseline (speedup 1.0000x reference)
import functools

import jax
import jax.numpy as jnp
from jax import lax
from jax.experimental import pallas as pl
from jax.experimental.pallas import tpu as pltpu
from jax.experimental.pallas import tpu_sc as plsc

B = 16384
D = 16
RPB = 8
WIDE = RPB * D
N_ROWS = 1000000
NC = 2
NS = 16
L = 16
NW = NC * NS
BPW = B // NW
CH = 128
NCH = BPW // CH
GPC = CH // L

_mesh = plsc.VectorSubcoreMesh(
    core_axis_name="c", subcore_axis_name="s", num_cores=NC, num_subcores=NS
)


@functools.partial(
    pl.kernel,
    out_type=jax.ShapeDtypeStruct((B,), jnp.float32),
    mesh=_mesh,
    scratch_types=[
        pltpu.VMEM((BPW,), jnp.int32),
        pltpu.VMEM((BPW,), jnp.int32),
        pltpu.VMEM((BPW,), jnp.int32),
        pltpu.VMEM((BPW,), jnp.int32),
        pltpu.VMEM((2, CH, WIDE), jnp.float32),
        pltpu.VMEM((2, CH, WIDE), jnp.float32),
        pltpu.VMEM((BPW,), jnp.float32),
        pltpu.SemaphoreType.DMA,
        pltpu.SemaphoreType.DMA,
        pltpu.SemaphoreType.DMA,
        pltpu.SemaphoreType.DMA,
    ],
    compiler_params=pltpu.CompilerParams(needs_layout_passes=False),
)
def _mf(v_hbm, j_hbm, hu_hbm, hi_hbm, out_hbm,
        vidx, jidx, vbun, jbun, hu_b, hi_b, out_v,
        sem_u0, sem_u1, sem_i0, sem_i1):
    wid = lax.axis_index("s") * NC + lax.axis_index("c")
    base = wid * BPW
    sems_u = (sem_u0, sem_u1)
    sems_i = (sem_i0, sem_i1)

    pltpu.sync_copy(v_hbm.at[pl.ds(base, BPW)], vidx)
    pltpu.sync_copy(j_hbm.at[pl.ds(base, BPW)], jidx)

    for g in range(BPW // L):
        off = g * L
        vbun[pl.ds(off, L)] = lax.shift_right_logical(vidx[pl.ds(off, L)], 3)
        jbun[pl.ds(off, L)] = lax.shift_right_logical(jidx[pl.ds(off, L)], 3)

    def fire(k):
        buf = k % 2
        cu = pltpu.async_copy(
            hu_hbm.at[vbun.at[pl.ds(k * CH, CH)]], hu_b.at[buf], sems_u[buf])
        ci = pltpu.async_copy(
            hi_hbm.at[jbun.at[pl.ds(k * CH, CH)]], hi_b.at[buf], sems_i[buf])
        return cu, ci

    lane = lax.iota(jnp.int32, L)
    pending = [fire(0), fire(1)]

    for k in range(NCH):
        buf = k % 2
        cu, ci = pending[k]
        cu.wait()
        ci.wait()
        for g in range(GPC):
            off = k * CH + g * L
            vv = vidx[pl.ds(off, L)]
            jv = jidx[pl.ds(off, L)]
            cu_col = (vv & 7) * D
            ci_col = (jv & 7) * D
            rows = g * L + lane
            acc = jnp.zeros((L,), jnp.float32)
            for d in range(D):
                hu = plsc.load_gather(hu_b.at[buf], [rows, cu_col + d])
                hi = plsc.load_gather(hi_b.at[buf], [rows, ci_col + d])
                acc = acc + hu * hi
            out_v[pl.ds(off, L)] = acc
        if k + 2 < NCH:
            pending.append(fire(k + 2))

    pltpu.sync_copy(out_v, out_hbm.at[pl.ds(base, BPW)])


def kernel(u, i, r, v, j, h_u, h_i):
    del u, i, r
    hu2 = h_u.reshape(N_ROWS // RPB, WIDE)
    hi2 = h_i.reshape(N_ROWS // RPB, WIDE)
    return _mf(v.astype(jnp.int32), j.astype(jnp.int32), hu2, hi2)


# speedup vs baseline: 6.4500x; 6.4500x over previous
import functools

import jax
import jax.numpy as jnp
from jax import lax
from jax.experimental import pallas as pl
from jax.experimental.pallas import tpu as pltpu
from jax.experimental.pallas import tpu_sc as plsc

B = 16384
D = 16
N_ROWS = 1000000
NC = 2
NS = 16
L = 16
NW = NC * NS
BPW = B // NW
CH = 16
NCH = BPW // CH
NPH = NCH * 2

_mesh = plsc.VectorSubcoreMesh(
    core_axis_name="c", subcore_axis_name="s", num_cores=NC, num_subcores=NS
)


@functools.partial(
    pl.kernel,
    out_type=jax.ShapeDtypeStruct((B,), jnp.float32),
    mesh=_mesh,
    scratch_types=[
        pltpu.VMEM((BPW,), jnp.int32),
        pltpu.VMEM((BPW,), jnp.int32),
        pltpu.VMEM((2, CH, 8, 128), jnp.float32),
        pltpu.VMEM((2, CH, 8, 128), jnp.float32),
        pltpu.VMEM((BPW,), jnp.float32),
        pltpu.SemaphoreType.DMA,
        pltpu.SemaphoreType.DMA,
        pltpu.SemaphoreType.DMA,
        pltpu.SemaphoreType.DMA,
    ],
    compiler_params=pltpu.CompilerParams(needs_layout_passes=False),
)
def _mf(v_hbm, j_hbm, hut_hbm, hit_hbm, out_hbm,
        vidx, jidx, tu, ti, out_v,
        sem_u0, sem_u1, sem_i0, sem_i1):
    wid = lax.axis_index("s") * NC + lax.axis_index("c")
    base = wid * BPW
    sems_u = (sem_u0, sem_u1)
    sems_i = (sem_i0, sem_i1)

    pltpu.sync_copy(v_hbm.at[pl.ds(base, BPW)], vidx)
    pltpu.sync_copy(j_hbm.at[pl.ds(base, BPW)], jidx)

    def fire(ci, db, buf):
        rv = lax.shift_right_logical(vidx[pl.ds(ci * CH, CH)], 7)
        rj = lax.shift_right_logical(jidx[pl.ds(ci * CH, CH)], 7)
        for c in range(CH):
            cv = pl.multiple_of(rv[c] * 128, 128)
            cj = pl.multiple_of(rj[c] * 128, 128)
            pltpu.async_copy(
                hut_hbm.at[pl.ds(db * 8, 8), pl.ds(cv, 128)],
                tu.at[buf, c], sems_u[buf])
            pltpu.async_copy(
                hit_hbm.at[pl.ds(db * 8, 8), pl.ds(cj, 128)],
                ti.at[buf, c], sems_i[buf])

    def drain(buf):
        dummy = hut_hbm.at[pl.ds(0, 8), pl.ds(0, 128)]
        for c in range(CH):
            pltpu.make_async_copy(dummy, tu.at[buf, c], sems_u[buf]).wait()
            pltpu.make_async_copy(dummy, ti.at[buf, c], sems_i[buf]).wait()

    lane = lax.iota(jnp.int32, L)

    def compute(ci, db, buf):
        lv = vidx[pl.ds(ci * CH, CH)] & 127
        lj = jidx[pl.ds(ci * CH, CH)] & 127
        acc = jnp.zeros((L,), jnp.float32)
        for d8 in range(8):
            d8v = jnp.full((L,), d8, jnp.int32)
            hu = plsc.load_gather(tu.at[buf], [lane, d8v, lv])
            hi = plsc.load_gather(ti.at[buf], [lane, d8v, lj])
            acc = acc + hu * hi
        if db == 0:
            out_v[pl.ds(ci * CH, CH)] = acc
        else:
            out_v[pl.ds(ci * CH, CH)] = out_v[pl.ds(ci * CH, CH)] + acc

    fire(0, 0, 0)

    def chunk_body(ci, carry):
        fire(ci, 1, 1)
        drain(0)
        compute(ci, 0, 0)

        @pl.when(ci + 1 < NCH)
        def _():
            fire(ci + 1, 0, 0)

        drain(1)
        compute(ci, 1, 1)
        return carry

    lax.fori_loop(0, NCH, chunk_body, 0)

    pltpu.sync_copy(out_v, out_hbm.at[pl.ds(base, BPW)])


def kernel(u, i, r, v, j, h_u, h_i):
    del u, i, r
    return _mf(v.astype(jnp.int32), j.astype(jnp.int32), h_u.T, h_i.T)


# speedup vs baseline: 6.4537x; 1.0006x over previous
import functools

import jax
import jax.numpy as jnp
from jax import lax
from jax.experimental import pallas as pl
from jax.experimental.pallas import tpu as pltpu
from jax.experimental.pallas import tpu_sc as plsc

B = 16384
D = 16
N_ROWS = 1000000
NC = 2
NS = 16
L = 16
NW = NC * NS
BPW = B // NW
CH = 8
NP = BPW // (2 * CH)

_mesh = plsc.VectorSubcoreMesh(
    core_axis_name="c", subcore_axis_name="s", num_cores=NC, num_subcores=NS
)


@functools.partial(
    pl.kernel,
    out_type=jax.ShapeDtypeStruct((B,), jnp.float32),
    mesh=_mesh,
    scratch_types=[
        pltpu.VMEM((BPW + L,), jnp.int32),
        pltpu.VMEM((BPW + L,), jnp.int32),
        pltpu.VMEM((2, CH, 16, 128), jnp.float32),
        pltpu.VMEM((2, CH, 16, 128), jnp.float32),
        pltpu.VMEM((BPW,), jnp.float32),
        pltpu.SemaphoreType.DMA,
        pltpu.SemaphoreType.DMA,
        pltpu.SemaphoreType.DMA,
        pltpu.SemaphoreType.DMA,
    ],
    compiler_params=pltpu.CompilerParams(needs_layout_passes=False),
)
def _mf(v_hbm, j_hbm, hut_hbm, hit_hbm, out_hbm,
        vidx, jidx, tu, ti, out_v,
        sem_u0, sem_u1, sem_i0, sem_i1):
    wid = lax.axis_index("s") * NC + lax.axis_index("c")
    base = wid * BPW
    sems_u = (sem_u0, sem_u1)
    sems_i = (sem_i0, sem_i1)

    pltpu.sync_copy(v_hbm.at[pl.ds(base, BPW)], vidx.at[pl.ds(0, BPW)])
    pltpu.sync_copy(j_hbm.at[pl.ds(base, BPW)], jidx.at[pl.ds(0, BPW)])

    lane = lax.iota(jnp.int32, L)
    cids = lane & (CH - 1)

    def fire(ci, buf):
        rv = lax.shift_right_logical(vidx[pl.ds(ci * CH, L)], 7)
        rj = lax.shift_right_logical(jidx[pl.ds(ci * CH, L)], 7)
        for c in range(CH):
            cv = pl.multiple_of(rv[c] * 128, 128)
            cj = pl.multiple_of(rj[c] * 128, 128)
            pltpu.async_copy(
                hut_hbm.at[pl.ds(0, 16), pl.ds(cv, 128)],
                tu.at[buf, c], sems_u[buf])
            pltpu.async_copy(
                hit_hbm.at[pl.ds(0, 16), pl.ds(cj, 128)],
                ti.at[buf, c], sems_i[buf])

    def drain(buf):
        dummy = hut_hbm.at[pl.ds(0, 16), pl.ds(0, 128)]
        for c in range(CH):
            pltpu.make_async_copy(dummy, tu.at[buf, c], sems_u[buf]).wait()
            pltpu.make_async_copy(dummy, ti.at[buf, c], sems_i[buf]).wait()

    def compute8(ci, buf):
        lv = plsc.load_gather(vidx, [ci * CH + cids]) & 127
        lj = plsc.load_gather(jidx, [ci * CH + cids]) & 127
        acc = jnp.zeros((L,), jnp.float32)
        for d in range(D):
            dv = jnp.full((L,), d, jnp.int32)
            hu = plsc.load_gather(tu.at[buf], [cids, dv, lv])
            hi = plsc.load_gather(ti.at[buf], [cids, dv, lj])
            acc = acc + hu * hi
        return acc

    fire(0, 0)

    def pair_body(p, carry):
        fire(2 * p + 1, 1)
        drain(0)
        acc_a = compute8(2 * p, 0)

        @pl.when(p + 1 < NP)
        def _():
            fire(2 * p + 2, 0)

        drain(1)
        acc_b = compute8(2 * p + 1, 1)
        out_v[pl.ds(p * L, L)] = jnp.where(lane < CH, acc_a, acc_b)
        return carry

    lax.fori_loop(0, NP, pair_body, 0)

    pltpu.sync_copy(out_v, out_hbm.at[pl.ds(base, BPW)])


def kernel(u, i, r, v, j, h_u, h_i):
    del u, i, r
    return _mf(v.astype(jnp.int32), j.astype(jnp.int32), h_u.T, h_i.T)
